# baseline (speedup 1.0000x reference)
import functools

import jax
import jax.numpy as jnp
import numpy as np
from jax import lax
from jax.experimental import pallas as pl
from jax.experimental.pallas import tpu as pltpu
from jax.experimental.pallas import tpu_sc as plsc

N = 10000
K = 32
E = N * K
D_EDGE = 16
IN1 = K * D_EDGE
H = 256
MID = 5
PAD = 16
ROWS = 1000
GRID = N // ROWS

CH = 80
CPB = 5

TCH = 2000


def _sc_mesh_info():
    info = plsc.get_sparse_core_info()
    mesh = plsc.VectorSubcoreMesh(core_axis_name="c", subcore_axis_name="s")
    return info, mesh, info.num_cores * info.num_subcores


def _transpose_call(eaT):
    info, mesh, nw = _sc_mesh_info()
    epw = E // nw
    nch = epw // TCH

    @functools.partial(
        pl.kernel,
        out_type=jax.ShapeDtypeStruct((E * D_EDGE // 128, 128), jnp.float32),
        mesh=mesh,
        scratch_types=[
            pltpu.VMEM((D_EDGE * TCH,), jnp.float32),
            pltpu.VMEM((TCH * D_EDGE // 128, 128), jnp.float32),
            pltpu.SemaphoreType.DMA,
        ],
        compiler_params=pltpu.CompilerParams(
            use_tc_tiling_on_sc=False, needs_layout_passes=False),
    )
    def tr_k(ea_hbm, out_hbm, in_v, out_v, sem):
        wid = lax.axis_index("s") * info.num_cores + lax.axis_index("c")
        lanes = lax.iota(jnp.int32, 16)

        def chunk(j, carry):
            col0 = wid * epw + j * TCH
            cps = [
                pltpu.async_copy(
                    ea_hbm.at[m, pl.ds(col0, TCH)],
                    in_v.at[pl.ds(m * TCH, TCH)],
                    sem,
                )
                for m in range(D_EDGE)
            ]
            for cp in cps:
                cp.wait()

            def group(g, carry2):
                for u in range(8):
                    col = g * 8 + u
                    vec = plsc.load_gather(in_v, [lanes * TCH + col])
                    out_v[g, pl.ds(u * 16, 16)] = vec
                return carry2

            lax.fori_loop(0, TCH // 8, group, 0)
            pltpu.sync_copy(
                out_v,
                out_hbm.at[pl.ds((wid * epw + j * TCH) // 8, TCH // 8), :],
            )
            return carry

        lax.fori_loop(0, nch, chunk, 0)

    return tr_k(eaT)


def _mlp1_call(A, w1r, b1, w2, b2, w3, b3, w4, b4):
    def body(h_ref, w1_ref, b1_ref, w2_ref, b2_ref, w3_ref, b3_ref,
             w4_ref, b4_ref, out_ref):
        x = h_ref[...].reshape(ROWS, IN1)
        h = jnp.tanh(jnp.dot(x, w1_ref[...]) + b1_ref[...])
        h = jnp.tanh(jnp.dot(h, w2_ref[...]) + b2_ref[...])
        h = jnp.tanh(jnp.dot(h, w3_ref[...]) + b3_ref[...])
        out_ref[...] = jax.nn.sigmoid(jnp.dot(h, w4_ref[...]) + b4_ref[...])

    full = lambda *s: pl.BlockSpec(s, lambda i: (0,) * len(s))
    return pl.pallas_call(
        body,
        grid=(GRID,),
        in_specs=[
            pl.BlockSpec((4 * ROWS, 128), lambda i: (i, 0)),
            full(IN1, H), full(1, H),
            full(H, H), full(1, H),
            full(H, H), full(1, H),
            full(H, PAD), full(1, PAD),
        ],
        out_specs=pl.BlockSpec((ROWS, PAD), lambda i: (i, 0)),
        out_shape=jax.ShapeDtypeStruct((N, PAD), jnp.float32),
    )(A, w1r, b1, w2, b2, w3, b3, w4, b4)


def _mlp2_body(h_ref, w1, b1, w2, b2, w3, b3, w4, b4, out_ref):
    h = jnp.tanh(jnp.dot(h_ref[...], w1[...]) + b1[...])
    h = jnp.tanh(jnp.dot(h, w2[...]) + b2[...])
    h = jnp.tanh(jnp.dot(h, w3[...]) + b3[...])
    out_ref[...] = jax.nn.sigmoid(jnp.dot(h, w4[...]) + b4[...])


def _mlp2_call(h, w1, b1, w2, b2, w3, b3, w4, b4):
    full = lambda r, c: pl.BlockSpec((r, c), lambda i: (0, 0))
    return pl.pallas_call(
        _mlp2_body,
        grid=(GRID,),
        in_specs=[
            pl.BlockSpec((ROWS, K * PAD), lambda i: (i, 0)),
            full(K * PAD, H), full(1, H),
            full(H, H), full(1, H),
            full(H, H), full(1, H),
            full(H, 1), full(1, 1),
        ],
        out_specs=pl.BlockSpec((ROWS, 1), lambda i: (i, 0)),
        out_shape=jax.ShapeDtypeStruct((N, 1), jnp.float32),
    )(h, w1, b1, w2, b2, w3, b3, w4, b4)


def _gather_call(table, edge_index):
    info, mesh, nw = _sc_mesh_info()
    ipw = E // nw
    bursts = ipw // (CPB * CH)

    @functools.partial(
        pl.kernel,
        out_type=jax.ShapeDtypeStruct((E, PAD), jnp.float32),
        mesh=mesh,
        scratch_types=[
            pltpu.VMEM((ipw,), jnp.int32),
            pltpu.VMEM((CPB * CH, PAD), jnp.float32),
            pltpu.SemaphoreType.DMA,
        ],
        compiler_params=pltpu.CompilerParams(use_tc_tiling_on_sc=False),
    )
    def gather_k(table_hbm, idx_hbm, out_hbm, idx_v, rows_v, sem):
        wid = lax.axis_index("s") * info.num_cores + lax.axis_index("c")
        pltpu.sync_copy(idx_hbm.at[1, pl.ds(wid * ipw, ipw)], idx_v)

        def burst(b, carry):
            cps = [
                pltpu.async_copy(
                    table_hbm.at[idx_v.at[pl.ds((b * CPB + j) * CH, CH)]],
                    rows_v.at[pl.ds(j * CH, CH)],
                    sem,
                )
                for j in range(CPB)
            ]
            for cp in cps:
                cp.wait()
            pltpu.sync_copy(
                rows_v,
                out_hbm.at[pl.ds(wid * ipw + b * (CPB * CH), CPB * CH)],
            )
            return carry

        lax.fori_loop(0, bursts, burst, 0)

    return gather_k(table, edge_index)


def kernel(x, edge_index, edge_attr,
           W1, b1, W2, b2, W3, b3, W4, b4,
           V1, c1, V2, c2, V3, c3, V4, c4):
    f32 = jnp.float32
    W4p = jnp.concatenate([W4, jnp.zeros((H, PAD - MID), f32)], axis=1)
    b4p = jnp.concatenate([b4, jnp.zeros((PAD - MID,), f32)])
    V1p = jnp.pad(V1.reshape(K, MID, H), ((0, 0), (0, PAD - MID), (0, 0)))
    V1p = V1p.reshape(K * PAD, H)

    r1 = lambda v: v.reshape(1, -1)

    A = _transpose_call(edge_attr.T)

    out1 = _mlp1_call(A, W1, r1(b1), W2, r1(b2), W3, r1(b3), W4p, r1(b4p))

    xj = _gather_call(out1, edge_index)

    h2 = xj.reshape(N, K * PAD)
    out2 = _mlp2_call(h2, V1p, r1(c1), V2, r1(c2), V3, r1(c3), V4, r1(c4))
    return jnp.squeeze(out2, 1)


# speedup vs baseline: 1.1666x; 1.1666x over previous
import functools

import jax
import jax.numpy as jnp
import numpy as np
from jax import lax
from jax.experimental import pallas as pl
from jax.experimental.pallas import tpu as pltpu
from jax.experimental.pallas import tpu_sc as plsc

N = 10000
K = 32
E = N * K
D_EDGE = 16
IN1 = K * D_EDGE
H = 256
MID = 5
PAD = 16
ROWS = 1000
GRID = N // ROWS

CH = 80
CPB = 5

TCH = 2000


def _sc_mesh_info():
    info = plsc.get_sparse_core_info()
    mesh = plsc.VectorSubcoreMesh(core_axis_name="c", subcore_axis_name="s")
    return info, mesh, info.num_cores * info.num_subcores


def _transpose_call(eaT):
    info, mesh, nw = _sc_mesh_info()
    epw = E // nw
    nch = epw // TCH

    @functools.partial(
        pl.kernel,
        out_type=jax.ShapeDtypeStruct((E * D_EDGE // 128, 128), jnp.float32),
        mesh=mesh,
        scratch_types=[
            pltpu.VMEM((D_EDGE * TCH,), jnp.float32),
            pltpu.VMEM((TCH * D_EDGE // 128, 128), jnp.float32),
            pltpu.SemaphoreType.DMA,
        ],
        compiler_params=pltpu.CompilerParams(
            use_tc_tiling_on_sc=False, needs_layout_passes=False),
    )
    def tr_k(ea_hbm, out_hbm, in_v, out_v, sem):
        wid = lax.axis_index("s") * info.num_cores + lax.axis_index("c")
        lanes = lax.iota(jnp.int32, 16)
        skews = [(lanes + d) % 16 for d in range(16)]
        gbases = [lanes * TCH + s for s in skews]
        rbases = [s // 8 for s in skews]
        cbases = [(s % 8) * 16 + lanes for s in skews]

        def chunk(j, carry):
            col0 = wid * epw + j * TCH
            cps = [
                pltpu.async_copy(
                    ea_hbm.at[m, pl.ds(col0, TCH)],
                    in_v.at[pl.ds(m * TCH, TCH)],
                    sem,
                )
                for m in range(D_EDGE)
            ]
            for cp in cps:
                cp.wait()

            def group(g, carry2):
                for d in range(16):
                    vec = plsc.load_gather(in_v, [gbases[d] + g * 16])
                    plsc.store_scatter(out_v, [rbases[d] + g * 2, cbases[d]], vec)
                return carry2

            lax.fori_loop(0, TCH // 16, group, 0)
            pltpu.sync_copy(
                out_v,
                out_hbm.at[pl.ds((wid * epw + j * TCH) // 8, TCH // 8), :],
            )
            return carry

        lax.fori_loop(0, nch, chunk, 0)

    return tr_k(eaT)


def _mlp1_call(A, w1r, b1, w2, b2, w3, b3, w4, b4):
    def body(h_ref, w1_ref, b1_ref, w2_ref, b2_ref, w3_ref, b3_ref,
             w4_ref, b4_ref, out_ref):
        x = h_ref[...].reshape(ROWS, IN1)
        h = jnp.tanh(jnp.dot(x, w1_ref[...]) + b1_ref[...])
        h = jnp.tanh(jnp.dot(h, w2_ref[...]) + b2_ref[...])
        h = jnp.tanh(jnp.dot(h, w3_ref[...]) + b3_ref[...])
        out_ref[...] = jax.nn.sigmoid(jnp.dot(h, w4_ref[...]) + b4_ref[...])

    full = lambda *s: pl.BlockSpec(s, lambda i: (0,) * len(s))
    return pl.pallas_call(
        body,
        grid=(GRID,),
        in_specs=[
            pl.BlockSpec((4 * ROWS, 128), lambda i: (i, 0)),
            full(IN1, H), full(1, H),
            full(H, H), full(1, H),
            full(H, H), full(1, H),
            full(H, PAD), full(1, PAD),
        ],
        out_specs=pl.BlockSpec((ROWS, PAD), lambda i: (i, 0)),
        out_shape=jax.ShapeDtypeStruct((N, PAD), jnp.float32),
    )(A, w1r, b1, w2, b2, w3, b3, w4, b4)


def _mlp2_body(h_ref, w1, b1, w2, b2, w3, b3, w4, b4, out_ref):
    h = jnp.tanh(jnp.dot(h_ref[...], w1[...]) + b1[...])
    h = jnp.tanh(jnp.dot(h, w2[...]) + b2[...])
    h = jnp.tanh(jnp.dot(h, w3[...]) + b3[...])
    out_ref[...] = jax.nn.sigmoid(jnp.dot(h, w4[...]) + b4[...])


def _mlp2_call(h, w1, b1, w2, b2, w3, b3, w4, b4):
    full = lambda r, c: pl.BlockSpec((r, c), lambda i: (0, 0))
    return pl.pallas_call(
        _mlp2_body,
        grid=(GRID,),
        in_specs=[
            pl.BlockSpec((ROWS, K * PAD), lambda i: (i, 0)),
            full(K * PAD, H), full(1, H),
            full(H, H), full(1, H),
            full(H, H), full(1, H),
            full(H, 1), full(1, 1),
        ],
        out_specs=pl.BlockSpec((ROWS, 1), lambda i: (i, 0)),
        out_shape=jax.ShapeDtypeStruct((N, 1), jnp.float32),
    )(h, w1, b1, w2, b2, w3, b3, w4, b4)


def _gather_call(table, edge_index):
    info, mesh, nw = _sc_mesh_info()
    ipw = E // nw
    bursts = ipw // (CPB * CH)

    @functools.partial(
        pl.kernel,
        out_type=jax.ShapeDtypeStruct((E, PAD), jnp.float32),
        mesh=mesh,
        scratch_types=[
            pltpu.VMEM((ipw,), jnp.int32),
            pltpu.VMEM((CPB * CH, PAD), jnp.float32),
            pltpu.SemaphoreType.DMA,
        ],
        compiler_params=pltpu.CompilerParams(use_tc_tiling_on_sc=False),
    )
    def gather_k(table_hbm, idx_hbm, out_hbm, idx_v, rows_v, sem):
        wid = lax.axis_index("s") * info.num_cores + lax.axis_index("c")
        pltpu.sync_copy(idx_hbm.at[1, pl.ds(wid * ipw, ipw)], idx_v)

        def burst(b, carry):
            cps = [
                pltpu.async_copy(
                    table_hbm.at[idx_v.at[pl.ds((b * CPB + j) * CH, CH)]],
                    rows_v.at[pl.ds(j * CH, CH)],
                    sem,
                )
                for j in range(CPB)
            ]
            for cp in cps:
                cp.wait()
            pltpu.sync_copy(
                rows_v,
                out_hbm.at[pl.ds(wid * ipw + b * (CPB * CH), CPB * CH)],
            )
            return carry

        lax.fori_loop(0, bursts, burst, 0)

    return gather_k(table, edge_index)


def kernel(x, edge_index, edge_attr,
           W1, b1, W2, b2, W3, b3, W4, b4,
           V1, c1, V2, c2, V3, c3, V4, c4):
    f32 = jnp.float32
    W4p = jnp.concatenate([W4, jnp.zeros((H, PAD - MID), f32)], axis=1)
    b4p = jnp.concatenate([b4, jnp.zeros((PAD - MID,), f32)])
    V1p = jnp.pad(V1.reshape(K, MID, H), ((0, 0), (0, PAD - MID), (0, 0)))
    V1p = V1p.reshape(K * PAD, H)

    r1 = lambda v: v.reshape(1, -1)

    A = _transpose_call(edge_attr.T)

    out1 = _mlp1_call(A, W1, r1(b1), W2, r1(b2), W3, r1(b3), W4p, r1(b4p))

    xj = _gather_call(out1, edge_index)

    h2 = xj.reshape(N, K * PAD)
    out2 = _mlp2_call(h2, V1p, r1(c1), V2, r1(c2), V3, r1(c3), V4, r1(c4))
    return jnp.squeeze(out2, 1)


# speedup vs baseline: 1.3121x; 1.1248x over previous
import functools

import jax
import jax.numpy as jnp
import numpy as np
from jax import lax
from jax.experimental import pallas as pl
from jax.experimental.pallas import tpu as pltpu
from jax.experimental.pallas import tpu_sc as plsc

N = 10000
K = 32
E = N * K
D_EDGE = 16
IN1 = K * D_EDGE
H = 256
MID = 5
PAD = 16
ROWS = 1000
GRID = N // ROWS

CH = 80
CPB = 5

TCH = 2000


def _sc_mesh_info():
    info = plsc.get_sparse_core_info()
    mesh = plsc.VectorSubcoreMesh(core_axis_name="c", subcore_axis_name="s")
    return info, mesh, info.num_cores * info.num_subcores


def _transpose_call(eaT):
    info, mesh, nw = _sc_mesh_info()
    epw = E // nw
    nch = epw // TCH

    @functools.partial(
        pl.kernel,
        out_type=jax.ShapeDtypeStruct((E * D_EDGE // 128, 128), jnp.float32),
        mesh=mesh,
        scratch_types=[
            pltpu.VMEM((D_EDGE * TCH,), jnp.float32),
            pltpu.VMEM((TCH * D_EDGE // 128, 128), jnp.float32),
            pltpu.SemaphoreType.DMA,
        ],
        compiler_params=pltpu.CompilerParams(
            use_tc_tiling_on_sc=False, needs_layout_passes=False),
    )
    def tr_k(ea_hbm, out_hbm, in_v, out_v, sem):
        wid = lax.axis_index("s") * info.num_cores + lax.axis_index("c")
        lanes = lax.iota(jnp.int32, 16)
        skews = [(lanes + d) % 16 for d in range(16)]
        gbases = [lanes * TCH + s for s in skews]
        rbases = [s // 8 for s in skews]
        cbases = [(s % 8) * 16 + lanes for s in skews]

        def chunk(j, carry):
            col0 = wid * epw + j * TCH
            cps = [
                pltpu.async_copy(
                    ea_hbm.at[m, pl.ds(col0, TCH)],
                    in_v.at[pl.ds(m * TCH, TCH)],
                    sem,
                )
                for m in range(D_EDGE)
            ]
            for cp in cps:
                cp.wait()

            def group(g, carry2):
                for d in range(16):
                    vec = plsc.load_gather(in_v, [gbases[d] + g * 16])
                    plsc.store_scatter(out_v, [rbases[d] + g * 2, cbases[d]], vec)
                return carry2

            lax.fori_loop(0, TCH // 16, group, 0)
            pltpu.sync_copy(
                out_v,
                out_hbm.at[pl.ds((wid * epw + j * TCH) // 8, TCH // 8), :],
            )
            return carry

        lax.fori_loop(0, nch, chunk, 0)

    return tr_k(eaT)


def _mlp1_call(A, w1r, b1, w2, b2, w3, b3, w4, b4):
    def body(h_ref, w1_ref, b1_ref, w2_ref, b2_ref, w3_ref, b3_ref,
             w4_ref, b4_ref, out_ref):
        x = h_ref[...].reshape(ROWS, IN1)
        h = jnp.tanh(jnp.dot(x, w1_ref[...]) + b1_ref[...])
        h = jnp.tanh(jnp.dot(h, w2_ref[...]) + b2_ref[...])
        h = jnp.tanh(jnp.dot(h, w3_ref[...]) + b3_ref[...])
        out_ref[...] = jax.nn.sigmoid(jnp.dot(h, w4_ref[...]) + b4_ref[...])

    full = lambda *s: pl.BlockSpec(s, lambda i: (0,) * len(s))
    return pl.pallas_call(
        body,
        grid=(GRID,),
        in_specs=[
            pl.BlockSpec((4 * ROWS, 128), lambda i: (i, 0)),
            full(IN1, H), full(1, H),
            full(H, H), full(1, H),
            full(H, H), full(1, H),
            full(H, PAD), full(1, PAD),
        ],
        out_specs=pl.BlockSpec((ROWS, PAD), lambda i: (i, 0)),
        out_shape=jax.ShapeDtypeStruct((N, PAD), jnp.float32),
    )(A, w1r, b1, w2, b2, w3, b3, w4, b4)


def _mlp2_body(h_ref, w1, b1, w2, b2, w3, b3, w4, b4, out_ref):
    h = jnp.tanh(jnp.dot(h_ref[...].reshape(ROWS, K * PAD), w1[...]) + b1[...])
    h = jnp.tanh(jnp.dot(h, w2[...]) + b2[...])
    h = jnp.tanh(jnp.dot(h, w3[...]) + b3[...])
    out_ref[...] = jax.nn.sigmoid(jnp.dot(h, w4[...]) + b4[...])


def _mlp2_call(h, w1, b1, w2, b2, w3, b3, w4, b4):
    full = lambda r, c: pl.BlockSpec((r, c), lambda i: (0, 0))
    return pl.pallas_call(
        _mlp2_body,
        grid=(GRID,),
        in_specs=[
            pl.BlockSpec((4 * ROWS, 128), lambda i: (i, 0)),
            full(K * PAD, H), full(1, H),
            full(H, H), full(1, H),
            full(H, H), full(1, H),
            full(H, 1), full(1, 1),
        ],
        out_specs=pl.BlockSpec((ROWS, 1), lambda i: (i, 0)),
        out_shape=jax.ShapeDtypeStruct((N, 1), jnp.float32),
    )(h, w1, b1, w2, b2, w3, b3, w4, b4)


def _gather_call(table, edge_index):
    info, mesh, nw = _sc_mesh_info()
    ipw = E // nw
    bursts = ipw // (CPB * CH)

    @functools.partial(
        pl.kernel,
        out_type=jax.ShapeDtypeStruct((E, PAD), jnp.float32),
        mesh=mesh,
        scratch_types=[
            pltpu.VMEM((ipw,), jnp.int32),
            pltpu.VMEM((CPB * CH, PAD), jnp.float32),
            pltpu.SemaphoreType.DMA,
        ],
        compiler_params=pltpu.CompilerParams(use_tc_tiling_on_sc=False),
    )
    def gather_k(table_hbm, idx_hbm, out_hbm, idx_v, rows_v, sem):
        wid = lax.axis_index("s") * info.num_cores + lax.axis_index("c")
        pltpu.sync_copy(idx_hbm.at[1, pl.ds(wid * ipw, ipw)], idx_v)

        def burst(b, carry):
            cps = [
                pltpu.async_copy(
                    table_hbm.at[idx_v.at[pl.ds((b * CPB + j) * CH, CH)]],
                    rows_v.at[pl.ds(j * CH, CH)],
                    sem,
                )
                for j in range(CPB)
            ]
            for cp in cps:
                cp.wait()
            pltpu.sync_copy(
                rows_v,
                out_hbm.at[pl.ds(wid * ipw + b * (CPB * CH), CPB * CH)],
            )
            return carry

        lax.fori_loop(0, bursts, burst, 0)

    return gather_k(table, edge_index)


def kernel(x, edge_index, edge_attr,
           W1, b1, W2, b2, W3, b3, W4, b4,
           V1, c1, V2, c2, V3, c3, V4, c4):
    f32 = jnp.float32
    W4p = jnp.concatenate([W4, jnp.zeros((H, PAD - MID), f32)], axis=1)
    b4p = jnp.concatenate([b4, jnp.zeros((PAD - MID,), f32)])
    V1p = jnp.pad(V1.reshape(K, MID, H), ((0, 0), (0, PAD - MID), (0, 0)))
    V1p = V1p.reshape(K * PAD, H)

    r1 = lambda v: v.reshape(1, -1)

    A = _transpose_call(edge_attr.T)

    out1 = _mlp1_call(A, W1, r1(b1), W2, r1(b2), W3, r1(b3), W4p, r1(b4p))

    xj = _gather_call(out1, edge_index)

    h2 = xj.reshape(K * PAD * N // 128, 128)
    out2 = _mlp2_call(h2, V1p, r1(c1), V2, r1(c2), V3, r1(c3), V4, r1(c4))
    return jnp.squeeze(out2, 1)


# speedup vs baseline: 1.3699x; 1.0440x over previous
import functools

import jax
import jax.numpy as jnp
import numpy as np
from jax import lax
from jax.experimental import pallas as pl
from jax.experimental.pallas import tpu as pltpu
from jax.experimental.pallas import tpu_sc as plsc

N = 10000
K = 32
E = N * K
D_EDGE = 16
IN1 = K * D_EDGE
H = 256
MID = 5
PAD = 16
ROWS = 1000
GRID = N // ROWS

CH = 80
CPB = 5

TCH = 2000


def _sc_mesh_info():
    info = plsc.get_sparse_core_info()
    mesh = plsc.VectorSubcoreMesh(core_axis_name="c", subcore_axis_name="s")
    return info, mesh, info.num_cores * info.num_subcores


def _transpose_call(eaT):
    info, mesh, nw = _sc_mesh_info()
    epw = E // nw
    nch = epw // TCH

    @functools.partial(
        pl.kernel,
        out_type=jax.ShapeDtypeStruct((E * D_EDGE // 128, 128), jnp.float32),
        mesh=mesh,
        scratch_types=[
            pltpu.VMEM((2 * D_EDGE * TCH,), jnp.float32),
            pltpu.VMEM((TCH * D_EDGE // 128, 128), jnp.float32),
            pltpu.SemaphoreType.DMA,
            pltpu.SemaphoreType.DMA,
        ],
        compiler_params=pltpu.CompilerParams(
            use_tc_tiling_on_sc=False, needs_layout_passes=False),
    )
    def tr_k(ea_hbm, out_hbm, in_v, out_v, sem0, sem1):
        wid = lax.axis_index("s") * info.num_cores + lax.axis_index("c")
        lanes = lax.iota(jnp.int32, 16)
        sems = (sem0, sem1)
        skews = [(lanes + d) % 16 for d in range(16)]
        gbases = [lanes * TCH + s for s in skews]
        rbases = [s // 8 for s in skews]
        cbases = [(s % 8) * 16 + lanes for s in skews]

        def issue(j):
            buf = j % 2
            col0 = wid * epw + j * TCH
            return [
                pltpu.async_copy(
                    ea_hbm.at[m, pl.ds(col0, TCH)],
                    in_v.at[pl.ds(buf * D_EDGE * TCH + m * TCH, TCH)],
                    sems[buf],
                )
                for m in range(D_EDGE)
            ]

        pend = issue(0)
        for j in range(nch):
            for cp in pend:
                cp.wait()
            if j + 1 < nch:
                pend = issue(j + 1)
            base = (j % 2) * D_EDGE * TCH

            def group(g, carry2, _base=base):
                for d in range(16):
                    vec = plsc.load_gather(in_v, [gbases[d] + (_base + g * 16)])
                    plsc.store_scatter(out_v, [rbases[d] + g * 2, cbases[d]], vec)
                return carry2

            lax.fori_loop(0, TCH // 16, group, 0)
            pltpu.sync_copy(
                out_v,
                out_hbm.at[pl.ds((wid * epw + j * TCH) // 8, TCH // 8), :],
            )

    return tr_k(eaT)


def _mlp1_call(A, w1r, b1, w2, b2, w3, b3, w4, b4):
    def body(h_ref, w1_ref, b1_ref, w2_ref, b2_ref, w3_ref, b3_ref,
             w4_ref, b4_ref, out_ref):
        x = h_ref[...].reshape(ROWS, IN1)
        h = jnp.tanh(jnp.dot(x, w1_ref[...]) + b1_ref[...])
        h = jnp.tanh(jnp.dot(h, w2_ref[...]) + b2_ref[...])
        h = jnp.tanh(jnp.dot(h, w3_ref[...]) + b3_ref[...])
        out_ref[...] = jax.nn.sigmoid(jnp.dot(h, w4_ref[...]) + b4_ref[...])

    full = lambda *s: pl.BlockSpec(s, lambda i: (0,) * len(s))
    return pl.pallas_call(
        body,
        grid=(GRID,),
        in_specs=[
            pl.BlockSpec((4 * ROWS, 128), lambda i: (i, 0)),
            full(IN1, H), full(1, H),
            full(H, H), full(1, H),
            full(H, H), full(1, H),
            full(H, PAD), full(1, PAD),
        ],
        out_specs=pl.BlockSpec((ROWS, PAD), lambda i: (i, 0)),
        out_shape=jax.ShapeDtypeStruct((N, PAD), jnp.float32),
    )(A, w1r, b1, w2, b2, w3, b3, w4, b4)


def _mlp2_body(h_ref, w1, b1, w2, b2, w3, b3, w4, b4, out_ref):
    h = jnp.tanh(jnp.dot(h_ref[...].reshape(ROWS, K * PAD), w1[...]) + b1[...])
    h = jnp.tanh(jnp.dot(h, w2[...]) + b2[...])
    h = jnp.tanh(jnp.dot(h, w3[...]) + b3[...])
    out_ref[...] = jax.nn.sigmoid(jnp.dot(h, w4[...]) + b4[...])


def _mlp2_call(h, w1, b1, w2, b2, w3, b3, w4, b4):
    full = lambda r, c: pl.BlockSpec((r, c), lambda i: (0, 0))
    return pl.pallas_call(
        _mlp2_body,
        grid=(GRID,),
        in_specs=[
            pl.BlockSpec((4 * ROWS, 128), lambda i: (i, 0)),
            full(K * PAD, H), full(1, H),
            full(H, H), full(1, H),
            full(H, H), full(1, H),
            full(H, 1), full(1, 1),
        ],
        out_specs=pl.BlockSpec((ROWS, 1), lambda i: (i, 0)),
        out_shape=jax.ShapeDtypeStruct((N, 1), jnp.float32),
    )(h, w1, b1, w2, b2, w3, b3, w4, b4)


def _gather_call(table, edge_index):
    info, mesh, nw = _sc_mesh_info()
    ipw = E // nw
    bursts = ipw // (CPB * CH)

    @functools.partial(
        pl.kernel,
        out_type=jax.ShapeDtypeStruct((E, PAD), jnp.float32),
        mesh=mesh,
        scratch_types=[
            pltpu.VMEM((ipw,), jnp.int32),
            pltpu.VMEM((CPB * CH, PAD), jnp.float32),
            pltpu.SemaphoreType.DMA,
        ],
        compiler_params=pltpu.CompilerParams(use_tc_tiling_on_sc=False),
    )
    def gather_k(table_hbm, idx_hbm, out_hbm, idx_v, rows_v, sem):
        wid = lax.axis_index("s") * info.num_cores + lax.axis_index("c")
        pltpu.sync_copy(idx_hbm.at[1, pl.ds(wid * ipw, ipw)], idx_v)

        def burst(b, carry):
            cps = [
                pltpu.async_copy(
                    table_hbm.at[idx_v.at[pl.ds((b * CPB + j) * CH, CH)]],
                    rows_v.at[pl.ds(j * CH, CH)],
                    sem,
                )
                for j in range(CPB)
            ]
            for cp in cps:
                cp.wait()
            pltpu.sync_copy(
                rows_v,
                out_hbm.at[pl.ds(wid * ipw + b * (CPB * CH), CPB * CH)],
            )
            return carry

        lax.fori_loop(0, bursts, burst, 0)

    return gather_k(table, edge_index)


def kernel(x, edge_index, edge_attr,
           W1, b1, W2, b2, W3, b3, W4, b4,
           V1, c1, V2, c2, V3, c3, V4, c4):
    f32 = jnp.float32
    W4p = jnp.concatenate([W4, jnp.zeros((H, PAD - MID), f32)], axis=1)
    b4p = jnp.concatenate([b4, jnp.zeros((PAD - MID,), f32)])
    V1p = jnp.pad(V1.reshape(K, MID, H), ((0, 0), (0, PAD - MID), (0, 0)))
    V1p = V1p.reshape(K * PAD, H)

    r1 = lambda v: v.reshape(1, -1)

    A = _transpose_call(edge_attr.T)

    out1 = _mlp1_call(A, W1, r1(b1), W2, r1(b2), W3, r1(b3), W4p, r1(b4p))

    xj = _gather_call(out1, edge_index)

    h2 = xj.reshape(K * PAD * N // 128, 128)
    out2 = _mlp2_call(h2, V1p, r1(c1), V2, r1(c2), V3, r1(c3), V4, r1(c4))
    return jnp.squeeze(out2, 1)


# speedup vs baseline: 1.3961x; 1.0191x over previous
import functools

import jax
import jax.numpy as jnp
import numpy as np
from jax import lax
from jax.experimental import pallas as pl
from jax.experimental.pallas import tpu as pltpu
from jax.experimental.pallas import tpu_sc as plsc

N = 10000
K = 32
E = N * K
D_EDGE = 16
IN1 = K * D_EDGE
H = 256
MID = 5
PAD = 16
ROWS = 1000
GRID = N // ROWS

CH = 80
CPB = 5

TCH = 2000


def _sc_mesh_info():
    info = plsc.get_sparse_core_info()
    mesh = plsc.VectorSubcoreMesh(core_axis_name="c", subcore_axis_name="s")
    return info, mesh, info.num_cores * info.num_subcores


def _transpose_call(eaT):
    info, mesh, nw = _sc_mesh_info()
    epw = E // nw
    nch = epw // TCH

    @functools.partial(
        pl.kernel,
        out_type=jax.ShapeDtypeStruct((E * D_EDGE // 128, 128), jnp.float32),
        mesh=mesh,
        scratch_types=[
            pltpu.VMEM((2 * D_EDGE * TCH,), jnp.float32),
            pltpu.VMEM((TCH * D_EDGE // 128, 128), jnp.float32),
            pltpu.SemaphoreType.DMA,
            pltpu.SemaphoreType.DMA,
        ],
        compiler_params=pltpu.CompilerParams(
            use_tc_tiling_on_sc=False, needs_layout_passes=False),
    )
    def tr_k(ea_hbm, out_hbm, in_v, out_v, sem0, sem1):
        wid = lax.axis_index("s") * info.num_cores + lax.axis_index("c")
        lanes = lax.iota(jnp.int32, 16)
        sems = (sem0, sem1)
        skews = [(lanes + d) % 16 for d in range(16)]
        gbases = [lanes * TCH + s for s in skews]
        rbases = [s // 8 for s in skews]
        cbases = [(s % 8) * 16 + lanes for s in skews]

        def issue(j):
            buf = j % 2
            col0 = wid * epw + j * TCH
            return [
                pltpu.async_copy(
                    ea_hbm.at[m, pl.ds(col0, TCH)],
                    in_v.at[pl.ds(buf * D_EDGE * TCH + m * TCH, TCH)],
                    sems[buf],
                )
                for m in range(D_EDGE)
            ]

        pend = issue(0)
        for j in range(nch):
            for cp in pend:
                cp.wait()
            if j + 1 < nch:
                pend = issue(j + 1)
            base = (j % 2) * D_EDGE * TCH

            def group(g, carry2, _base=base):
                for d in range(16):
                    vec = plsc.load_gather(in_v, [gbases[d] + (_base + g * 16)])
                    plsc.store_scatter(out_v, [rbases[d] + g * 2, cbases[d]], vec)
                return carry2

            lax.fori_loop(0, TCH // 16, group, 0)
            pltpu.sync_copy(
                out_v,
                out_hbm.at[pl.ds((wid * epw + j * TCH) // 8, TCH // 8), :],
            )

    return tr_k(eaT)


def _mlp1_call(A, w1r, b1, w2, b2, w3, b3, w4, b4):
    def body(h_ref, w1_ref, b1_ref, w2_ref, b2_ref, w3_ref, b3_ref,
             w4_ref, b4_ref, out_ref):
        x = h_ref[...].reshape(ROWS, IN1)
        h = jnp.tanh(jnp.dot(x, w1_ref[...]) + b1_ref[...])
        h = jnp.tanh(jnp.dot(h, w2_ref[...]) + b2_ref[...])
        h = jnp.tanh(jnp.dot(h, w3_ref[...]) + b3_ref[...])
        out_ref[...] = jax.nn.sigmoid(jnp.dot(h, w4_ref[...]) + b4_ref[...])

    full = lambda *s: pl.BlockSpec(s, lambda i: (0,) * len(s))
    return pl.pallas_call(
        body,
        grid=(GRID,),
        in_specs=[
            pl.BlockSpec((4 * ROWS, 128), lambda i: (i, 0)),
            full(IN1, H), full(1, H),
            full(H, H), full(1, H),
            full(H, H), full(1, H),
            full(H, PAD), full(1, PAD),
        ],
        out_specs=pl.BlockSpec((ROWS, PAD), lambda i: (i, 0)),
        out_shape=jax.ShapeDtypeStruct((N, PAD), jnp.float32),
    )(A, w1r, b1, w2, b2, w3, b3, w4, b4)


def _mlp2_body(h_ref, w1, b1, w2, b2, w3, b3, w4, b4, out_ref):
    h = jnp.tanh(jnp.dot(h_ref[...].reshape(ROWS, K * PAD), w1[...]) + b1[...])
    h = jnp.tanh(jnp.dot(h, w2[...]) + b2[...])
    h = jnp.tanh(jnp.dot(h, w3[...]) + b3[...])
    out_ref[...] = jax.nn.sigmoid(jnp.dot(h, w4[...]) + b4[...])


def _mlp2_call(h, w1, b1, w2, b2, w3, b3, w4, b4):
    full = lambda r, c: pl.BlockSpec((r, c), lambda i: (0, 0))
    return pl.pallas_call(
        _mlp2_body,
        grid=(GRID,),
        in_specs=[
            pl.BlockSpec((4 * ROWS, 128), lambda i: (i, 0)),
            full(K * PAD, H), full(1, H),
            full(H, H), full(1, H),
            full(H, H), full(1, H),
            full(H, 1), full(1, 1),
        ],
        out_specs=pl.BlockSpec((ROWS, 1), lambda i: (i, 0)),
        out_shape=jax.ShapeDtypeStruct((N, 1), jnp.float32),
    )(h, w1, b1, w2, b2, w3, b3, w4, b4)


def _gather_call(table, edge_index):
    info, mesh, nw = _sc_mesh_info()
    ipw = E // nw
    bursts = ipw // (CPB * CH)

    @functools.partial(
        pl.kernel,
        out_type=jax.ShapeDtypeStruct((E, PAD), jnp.float32),
        mesh=mesh,
        scratch_types=[
            pltpu.VMEM((ipw,), jnp.int32),
            pltpu.VMEM((2 * CPB * CH, PAD), jnp.float32),
            pltpu.SemaphoreType.DMA,
            pltpu.SemaphoreType.DMA,
            pltpu.SemaphoreType.DMA,
        ],
        compiler_params=pltpu.CompilerParams(use_tc_tiling_on_sc=False),
    )
    def gather_k(table_hbm, idx_hbm, out_hbm, idx_v, rows_v, sem, st0, st1):
        wid = lax.axis_index("s") * info.num_cores + lax.axis_index("c")
        sts = (st0, st1)
        BR = CPB * CH
        pltpu.sync_copy(idx_hbm.at[1, pl.ds(wid * ipw, ipw)], idx_v)

        def burst(b, carry):
            half = b % 2
            base = half * BR
            slab = out_hbm.at[pl.ds(wid * ipw + b * BR, BR)]

            @pl.when(b >= 2)
            def _drain():
                for h in range(2):
                    @pl.when(half == h)
                    def _w():
                        pltpu.make_async_copy(
                            rows_v.at[pl.ds(h * BR, BR)], slab, sts[h]
                        ).wait()

            cps = [
                pltpu.async_copy(
                    table_hbm.at[idx_v.at[pl.ds((b * CPB + j) * CH, CH)]],
                    rows_v.at[pl.ds(base + j * CH, CH)],
                    sem,
                )
                for j in range(CPB)
            ]
            for cp in cps:
                cp.wait()
            for h in range(2):
                @pl.when(half == h)
                def _st():
                    pltpu.async_copy(rows_v.at[pl.ds(h * BR, BR)], slab, sts[h])
            return carry

        lax.fori_loop(0, bursts, burst, 0)
        for h in range(2):
            pltpu.make_async_copy(
                rows_v.at[pl.ds(h * BR, BR)],
                out_hbm.at[pl.ds(wid * ipw, BR)],
                sts[h],
            ).wait()

    return gather_k(table, edge_index)


def kernel(x, edge_index, edge_attr,
           W1, b1, W2, b2, W3, b3, W4, b4,
           V1, c1, V2, c2, V3, c3, V4, c4):
    f32 = jnp.float32
    W4p = jnp.concatenate([W4, jnp.zeros((H, PAD - MID), f32)], axis=1)
    b4p = jnp.concatenate([b4, jnp.zeros((PAD - MID,), f32)])
    V1p = jnp.pad(V1.reshape(K, MID, H), ((0, 0), (0, PAD - MID), (0, 0)))
    V1p = V1p.reshape(K * PAD, H)

    r1 = lambda v: v.reshape(1, -1)

    A = _transpose_call(edge_attr.T)

    out1 = _mlp1_call(A, W1, r1(b1), W2, r1(b2), W3, r1(b3), W4p, r1(b4p))

    xj = _gather_call(out1, edge_index)

    h2 = xj.reshape(K * PAD * N // 128, 128)
    out2 = _mlp2_call(h2, V1p, r1(c1), V2, r1(c2), V3, r1(c3), V4, r1(c4))
    return jnp.squeeze(out2, 1)
